# baseline (speedup 1.0000x reference)
import functools

import jax
import jax.numpy as jnp
from jax import lax
from jax.experimental import pallas as pl
from jax.experimental.pallas import tpu as pltpu
from jax.experimental.pallas import tpu_sc as plsc

B = 2048
NFEAT = 128
EMB = 64
NCLASS = 40
ND0, ND1 = 16, 8
NB0, NB1 = 8, 4
R = 128

_HI = jax.lax.Precision.HIGHEST


def _dot(a, b):
    return jax.lax.dot_general(a, b, (((1,), (0,)), ((), ())),
                               preferred_element_type=jnp.float32,
                               precision=_HI)


def _tc_main_body(x0_ref, x1_ref, m2_ref,
                  wt_ref, w1a_ref, b1a_ref, w1b_ref, b1b_ref,
                  xs0_ref, h0_ref, h_ref):
    wt = wt_ref[...]
    w1a = w1a_ref[...]
    w1b = w1b_ref[...]
    b1a = b1a_ref[...]
    b1b = b1b_ref[...]

    wa1 = _dot(wt, w1a[:EMB])
    wa2 = _dot(wt, w1a[EMB:])

    x1_3 = x1_ref[...]
    m1 = jnp.sum(x1_3, axis=1) * (1.0 / ND0)
    m2 = m2_ref[...] * (1.0 / ND1)
    x1f = x1_3.reshape(R * ND0, NFEAT)
    h1 = jax.nn.relu(_dot(x1f, wa1) + _dot(m2, wa2) + b1a)
    p1 = jnp.sum(h1.reshape(R, ND0, EMB), axis=1) * (1.0 / ND0)
    x0b = x0_ref[...]
    xs0_ref[...] = _dot(x0b, wt)
    h0 = jax.nn.relu(_dot(x0b, wa1) + _dot(m1, wa2) + b1a)
    h0_ref[...] = h0
    h_ref[...] = _dot(h0, w1b[:EMB]) + _dot(p1, w1b[EMB:]) + b1b


def _tc_bi_body(xbi0_ref, m4_ref, e_ref, par_ref, xs0_ref, h0_ref, h_ref,
                wt_ref, w2a_ref, b2a_ref, w2b_ref, b2b_ref,
                l1w_ref, l1b_ref, l2w_ref, l2b_ref, out_ref):
    wt = wt_ref[...]
    w2a = w2a_ref[...]
    w2b = w2b_ref[...]
    b2a = b2a_ref[...]
    b2b = b2b_ref[...]
    wc1 = _dot(wt, w2a[:EMB])
    wc2 = _dot(wt, w2a[EMB:])
    xs0 = xs0_ref[...]
    h0 = h0_ref[...]
    h = h_ref[...]

    pk = e_ref[...]
    par = (par_ref[...] & 1).astype(jnp.float32)
    el = pk[:, :EMB]
    er = pk[:, EMB:]
    ef = el + par * (er - el)
    s = jnp.sum(ef.reshape(R, NB0, EMB), axis=1) * (1.0 / NB0)
    m4 = m4_ref[...] * (1.0 / NB1)
    g1 = jax.nn.relu(_dot(ef, w2a[:EMB]) + _dot(m4, wc2) + b2a)
    p2 = jnp.sum(g1.reshape(R, NB0, EMB), axis=1) * (1.0 / NB0)
    xbi0b = xbi0_ref[...]
    g0 = jax.nn.relu(_dot(xbi0b, wc1) + _dot(s, w2a[EMB:]) + b2a)
    g = _dot(g0, w2b[:EMB]) + _dot(p2, w2b[EMB:]) + b2b

    xl = [xs0, h0, h]
    yl = [g0, g]
    cross = [xi * yj for xi in xl for yj in yl] + xl
    l1w = l1w_ref[...]
    l1b = l1b_ref[0, 0]
    logits = [jnp.sum(c * l1w, axis=1, keepdims=True) + l1b for c in cross]
    amax = logits[0]
    for a in logits[1:]:
        amax = jnp.maximum(amax, a)
    es = [jnp.exp(a - amax) for a in logits]
    z = es[0]
    for ev in es[1:]:
        z = z + ev
    inv_z = 1.0 / z
    hidden = es[0] * inv_z * cross[0]
    for ev, c in zip(es[1:], cross[1:]):
        hidden = hidden + ev * inv_z * c

    out = _dot(hidden, l2w_ref[...]) + l2b_ref[...]
    omax = jnp.max(out, axis=1, keepdims=True)
    sh = out - omax
    lse = jnp.log(jnp.sum(jnp.exp(sh), axis=1, keepdims=True))
    out_ref[...] = sh - lse


def _full(shape):
    return pl.BlockSpec(shape, lambda i: (0,) * len(shape))


def _tc_main(x0, x1v, m2v, wt, w1a, b1a, w1b, b1b):
    grid_spec = pl.GridSpec(
        grid=(B // R,),
        in_specs=[
            pl.BlockSpec((R, NFEAT), lambda i: (i, 0)),
            pl.BlockSpec((R, ND0, NFEAT), lambda i: (i, 0, 0)),
            pl.BlockSpec((R * ND0, NFEAT), lambda i: (i, 0)),
            _full((NFEAT, EMB)),
            _full((2 * EMB, EMB)), _full((1, EMB)),
            _full((2 * EMB, EMB)), _full((1, EMB)),
        ],
        out_specs=[pl.BlockSpec((R, EMB), lambda i: (i, 0))] * 3,
    )
    return pl.pallas_call(
        _tc_main_body,
        grid_spec=grid_spec,
        out_shape=[jax.ShapeDtypeStruct((B, EMB), jnp.float32)] * 3,
        compiler_params=pltpu.CompilerParams(
            dimension_semantics=("arbitrary",)),
    )(x0, x1v, m2v, wt, w1a, b1a, w1b, b1b)


def _tc_bi(xbi0, m4v, ev, parv, xs0, h0, h, wt,
           w2a, b2a, w2b, b2b, l1w, l1b, l2w, l2b):
    grid_spec = pl.GridSpec(
        grid=(B // R,),
        in_specs=[
            pl.BlockSpec((R, NFEAT), lambda i: (i, 0)),
            pl.BlockSpec((R * NB0, NFEAT), lambda i: (i, 0)),
            pl.BlockSpec((R * NB0, NFEAT), lambda i: (i, 0)),
            pl.BlockSpec((R * NB0, 1), lambda i: (i, 0)),
            pl.BlockSpec((R, EMB), lambda i: (i, 0)),
            pl.BlockSpec((R, EMB), lambda i: (i, 0)),
            pl.BlockSpec((R, EMB), lambda i: (i, 0)),
            _full((NFEAT, EMB)),
            _full((2 * EMB, EMB)), _full((1, EMB)),
            _full((2 * EMB, EMB)), _full((1, EMB)),
            _full((1, EMB)), _full((1, 1)),
            _full((EMB, NCLASS)), _full((1, NCLASS)),
        ],
        out_specs=pl.BlockSpec((R, NCLASS), lambda i: (i, 0)),
    )
    return pl.pallas_call(
        _tc_bi_body,
        grid_spec=grid_spec,
        out_shape=jax.ShapeDtypeStruct((B, NCLASS), jnp.float32),
        compiler_params=pltpu.CompilerParams(
            dimension_semantics=("arbitrary",)),
    )(xbi0, m4v, ev, parv, xs0, h0, h, wt,
      w2a, b2a, w2b, b2b, l1w, l1b, l2w, l2b)


_NC, _NS = 2, 16
_NW = _NC * _NS
_NIDX = B * NB0
_BPW = _NIDX // _NW


def _sc_gather(table2, idx):
    mesh = plsc.VectorSubcoreMesh(core_axis_name="c", subcore_axis_name="s")

    @functools.partial(
        pl.kernel, mesh=mesh,
        out_type=jax.ShapeDtypeStruct((_NIDX, 2 * EMB), jnp.float32),
        scratch_types=[
            pltpu.VMEM((_BPW,), jnp.int32),
            pltpu.VMEM((_BPW,), jnp.int32),
            pltpu.VMEM((_BPW, 2 * EMB), jnp.float32),
            pltpu.SemaphoreType.DMA,
        ],
    )
    def k(table_hbm, idx_hbm, out_hbm, idx_v, idx2_v, rows_v, sem):
        wid = lax.axis_index("s") * _NC + lax.axis_index("c")
        base = wid * _BPW
        pltpu.sync_copy(idx_hbm.at[pl.ds(base, _BPW)], idx_v)
        for j in range(_BPW // 16):
            sl = pl.ds(j * 16, 16)
            idx2_v[sl] = lax.shift_right_logical(idx_v[sl], 1)
        pltpu.async_copy(table_hbm.at[idx2_v], rows_v, sem).wait()
        pltpu.sync_copy(rows_v, out_hbm.at[pl.ds(base, _BPW)])

    return k(table2, idx)


def _sc_group_sum(x, G):
    N = x.shape[0]
    CH = 256
    rows_pt = N // _NW
    nchunk = rows_pt // CH
    opc = CH // G
    mesh = plsc.VectorSubcoreMesh(core_axis_name="c", subcore_axis_name="s")

    @functools.partial(
        pl.kernel, mesh=mesh,
        out_type=jax.ShapeDtypeStruct((N // G, NFEAT), jnp.float32),
        scratch_types=[
            pltpu.VMEM((CH, NFEAT), jnp.float32),
            pltpu.VMEM((CH, NFEAT), jnp.float32),
            pltpu.VMEM((opc, NFEAT), jnp.float32),
            pltpu.VMEM((opc, NFEAT), jnp.float32),
            pltpu.SemaphoreType.DMA,
            pltpu.SemaphoreType.DMA,
            pltpu.SemaphoreType.DMA,
            pltpu.SemaphoreType.DMA,
        ],
    )
    def k(x_hbm, o_hbm, b0, b1, ob0, ob1, s0, s1, so0, so1):
        wid = lax.axis_index("s") * _NC + lax.axis_index("c")
        base = wid * rows_pt
        obase = wid * (rows_pt // G)
        bufs = ((b0, ob0, s0, so0), (b1, ob1, s1, so1))
        pltpu.async_copy(x_hbm.at[pl.ds(base, CH)], b0, s0)
        if nchunk > 1:
            pltpu.async_copy(x_hbm.at[pl.ds(base + CH, CH)], b1, s1)

        def step(i, carry):
            for bsel in range(2):
                buf, ob, sem, osem = bufs[bsel]
                c = 2 * i + bsel
                pltpu.make_async_copy(x_hbm.at[pl.ds(0, CH)], buf, sem).wait()

                def row_body(o, carry2):
                    for cc in range(NFEAT // 16):
                        sl = pl.ds(cc * 16, 16)
                        acc = buf[G * o, sl]
                        for j in range(1, G):
                            acc = acc + buf[G * o + j, sl]
                        ob[o, sl] = acc
                    return carry2

                @pl.when(c >= 2)
                def _():
                    pltpu.make_async_copy(
                        o_hbm.at[pl.ds(0, opc)], ob, osem).wait()

                lax.fori_loop(0, opc, row_body, 0, unroll=8)
                pltpu.async_copy(
                    ob, o_hbm.at[pl.ds(obase + c * opc, opc)], osem)

                @pl.when(c + 2 < nchunk)
                def _():
                    pltpu.async_copy(
                        x_hbm.at[pl.ds(base + (c + 2) * CH, CH)], buf, sem)
            return carry

        lax.fori_loop(0, nchunk // 2, step, 0, unroll=False)
        for bsel in range(2):
            _, ob, _, osem = bufs[bsel]
            pltpu.make_async_copy(o_hbm.at[pl.ds(0, opc)], ob, osem).wait()

    return k(x)


def kernel(x0, x1, x2, xbi0, xbi2, weight_trans, table, W1a, b1a, W1b, b1b,
           W2a, b2a, W2b, b2b, lin1_w, lin1_b, lin2_w, lin2_b, xbi1):
    m2s = _sc_group_sum(x2, ND1)
    e = _sc_gather(table.reshape(-1, 2 * EMB), xbi1)
    m4s = _sc_group_sum(xbi2, NB1)
    xs0, h0, h = _tc_main(
        x0,
        x1.reshape(B, ND0, NFEAT),
        m2s,
        weight_trans,
        W1a, b1a.reshape(1, EMB),
        W1b, b1b.reshape(1, EMB),
    )
    return _tc_bi(
        xbi0,
        m4s,
        e,
        xbi1.reshape(_NIDX, 1),
        xs0, h0, h,
        weight_trans,
        W2a, b2a.reshape(1, EMB),
        W2b, b2b.reshape(1, EMB),
        lin1_w.reshape(1, EMB), lin1_b.reshape(1, 1),
        lin2_w, lin2_b.reshape(1, NCLASS),
    )


# speedup vs baseline: 1.2030x; 1.2030x over previous
import functools

import jax
import jax.numpy as jnp
from jax import lax
from jax.experimental import pallas as pl
from jax.experimental.pallas import tpu as pltpu
from jax.experimental.pallas import tpu_sc as plsc

B = 2048
NFEAT = 128
EMB = 64
NCLASS = 40
ND0, ND1 = 16, 8
NB0, NB1 = 8, 4
R = 128
RA = 1024

_HI = jax.lax.Precision.HIGHEST


def _dot(a, b):
    return jax.lax.dot_general(a, b, (((1,), (0,)), ((), ())),
                               preferred_element_type=jnp.float32,
                               precision=_HI)


def _tc_main_a_body(x0_ref, x1_ref, x2_ref,
                    wt_ref, w1a_ref, b1a_ref, w1b_ref, b1b_ref,
                    xs0_ref, h0_ref, h_ref):
    wt = wt_ref[...]
    w1a = w1a_ref[...]
    w1b = w1b_ref[...]
    b1a = b1a_ref[...]
    b1b = b1b_ref[...]
    wa1 = _dot(wt, w1a[:EMB])
    wa2 = _dot(wt, w1a[EMB:])
    x1_3 = x1_ref[...]
    m1 = jnp.sum(x1_3, axis=1) * (1.0 / ND0)
    m2 = jnp.sum(x2_ref[...], axis=1) * (1.0 / ND1)
    x1f = x1_3.reshape(R * ND0, NFEAT)
    h1 = jax.nn.relu(_dot(x1f, wa1) + _dot(m2, wa2) + b1a)
    p1 = jnp.sum(h1.reshape(R, ND0, EMB), axis=1) * (1.0 / ND0)
    x0b = x0_ref[...]
    xs0_ref[...] = _dot(x0b, wt)
    h0 = jax.nn.relu(_dot(x0b, wa1) + _dot(m1, wa2) + b1a)
    h0_ref[...] = h0
    h_ref[...] = _dot(h0, w1b[:EMB]) + _dot(p1, w1b[EMB:]) + b1b


def _tc_main_body(x0_ref, x1_ref, m2_ref,
                  wt_ref, w1a_ref, b1a_ref, w1b_ref, b1b_ref,
                  xs0_ref, h0_ref, h_ref):
    wt = wt_ref[...]
    w1a = w1a_ref[...]
    w1b = w1b_ref[...]
    b1a = b1a_ref[...]
    b1b = b1b_ref[...]

    wa1 = _dot(wt, w1a[:EMB])
    wa2 = _dot(wt, w1a[EMB:])

    x1_3 = x1_ref[...]
    m1 = jnp.sum(x1_3, axis=1) * (1.0 / ND0)
    m2 = m2_ref[...] * (1.0 / ND1)
    x1f = x1_3.reshape(R * ND0, NFEAT)
    h1 = jax.nn.relu(_dot(x1f, wa1) + _dot(m2, wa2) + b1a)
    p1 = jnp.sum(h1.reshape(R, ND0, EMB), axis=1) * (1.0 / ND0)
    x0b = x0_ref[...]
    xs0_ref[...] = _dot(x0b, wt)
    h0 = jax.nn.relu(_dot(x0b, wa1) + _dot(m1, wa2) + b1a)
    h0_ref[...] = h0
    h_ref[...] = _dot(h0, w1b[:EMB]) + _dot(p1, w1b[EMB:]) + b1b


def _tc_bi_body(xbi0_ref, m4_ref, e_ref, par_ref, xs0_ref, h0_ref, h_ref,
                wt_ref, w2a_ref, b2a_ref, w2b_ref, b2b_ref,
                l1w_ref, l1b_ref, l2w_ref, l2b_ref, out_ref):
    wt = wt_ref[...]
    w2a = w2a_ref[...]
    w2b = w2b_ref[...]
    b2a = b2a_ref[...]
    b2b = b2b_ref[...]
    wc1 = _dot(wt, w2a[:EMB])
    wc2 = _dot(wt, w2a[EMB:])
    xs0 = xs0_ref[...]
    h0 = h0_ref[...]
    h = h_ref[...]

    pk = e_ref[...]
    par = (par_ref[...] & 1).astype(jnp.float32)
    el = pk[:, :EMB]
    er = pk[:, EMB:]
    ef = el + par * (er - el)
    s = jnp.sum(ef.reshape(R, NB0, EMB), axis=1) * (1.0 / NB0)
    m4 = m4_ref[...] * (1.0 / NB1)
    g1 = jax.nn.relu(_dot(ef, w2a[:EMB]) + _dot(m4, wc2) + b2a)
    p2 = jnp.sum(g1.reshape(R, NB0, EMB), axis=1) * (1.0 / NB0)
    xbi0b = xbi0_ref[...]
    g0 = jax.nn.relu(_dot(xbi0b, wc1) + _dot(s, w2a[EMB:]) + b2a)
    g = _dot(g0, w2b[:EMB]) + _dot(p2, w2b[EMB:]) + b2b

    xl = [xs0, h0, h]
    yl = [g0, g]
    cross = [xi * yj for xi in xl for yj in yl] + xl
    l1w = l1w_ref[...]
    l1b = l1b_ref[0, 0]
    logits = [jnp.sum(c * l1w, axis=1, keepdims=True) + l1b for c in cross]
    amax = logits[0]
    for a in logits[1:]:
        amax = jnp.maximum(amax, a)
    es = [jnp.exp(a - amax) for a in logits]
    z = es[0]
    for ev in es[1:]:
        z = z + ev
    inv_z = 1.0 / z
    hidden = es[0] * inv_z * cross[0]
    for ev, c in zip(es[1:], cross[1:]):
        hidden = hidden + ev * inv_z * c

    out = _dot(hidden, l2w_ref[...]) + l2b_ref[...]
    omax = jnp.max(out, axis=1, keepdims=True)
    sh = out - omax
    lse = jnp.log(jnp.sum(jnp.exp(sh), axis=1, keepdims=True))
    out_ref[...] = sh - lse


def _full(shape):
    return pl.BlockSpec(shape, lambda i: (0,) * len(shape))


def _tc_main_a(x0, x1v, x2v, wt, w1a, b1a, w1b, b1b):
    grid_spec = pl.GridSpec(
        grid=(RA // R,),
        in_specs=[
            pl.BlockSpec((R, NFEAT), lambda i: (i, 0)),
            pl.BlockSpec((R, ND0, NFEAT), lambda i: (i, 0, 0)),
            pl.BlockSpec((R * ND0, ND1, NFEAT), lambda i: (i, 0, 0)),
            _full((NFEAT, EMB)),
            _full((2 * EMB, EMB)), _full((1, EMB)),
            _full((2 * EMB, EMB)), _full((1, EMB)),
        ],
        out_specs=[pl.BlockSpec((R, EMB), lambda i: (i, 0))] * 3,
    )
    return pl.pallas_call(
        _tc_main_a_body,
        grid_spec=grid_spec,
        out_shape=[jax.ShapeDtypeStruct((RA, EMB), jnp.float32)] * 3,
        compiler_params=pltpu.CompilerParams(
            dimension_semantics=("arbitrary",)),
    )(x0, x1v, x2v, wt, w1a, b1a, w1b, b1b)


def _tc_main_b(x0, x1v, m2v, wt, w1a, b1a, w1b, b1b):
    off = RA // R
    grid_spec = pl.GridSpec(
        grid=((B - RA) // R,),
        in_specs=[
            pl.BlockSpec((R, NFEAT), lambda i: (i + off, 0)),
            pl.BlockSpec((R, ND0, NFEAT), lambda i: (i + off, 0, 0)),
            pl.BlockSpec((R * ND0, NFEAT), lambda i: (i, 0)),
            _full((NFEAT, EMB)),
            _full((2 * EMB, EMB)), _full((1, EMB)),
            _full((2 * EMB, EMB)), _full((1, EMB)),
        ],
        out_specs=[pl.BlockSpec((R, EMB), lambda i: (i, 0))] * 3,
    )
    return pl.pallas_call(
        _tc_main_body,
        grid_spec=grid_spec,
        out_shape=[jax.ShapeDtypeStruct((B - RA, EMB), jnp.float32)] * 3,
        compiler_params=pltpu.CompilerParams(
            dimension_semantics=("arbitrary",)),
    )(x0, x1v, m2v, wt, w1a, b1a, w1b, b1b)


def _tc_bi(xbi0, m4v, ev, parv, xs0, h0, h, wt,
           w2a, b2a, w2b, b2b, l1w, l1b, l2w, l2b):
    grid_spec = pl.GridSpec(
        grid=(B // R,),
        in_specs=[
            pl.BlockSpec((R, NFEAT), lambda i: (i, 0)),
            pl.BlockSpec((R * NB0, NFEAT), lambda i: (i, 0)),
            pl.BlockSpec((R * NB0, NFEAT), lambda i: (i, 0)),
            pl.BlockSpec((R * NB0, 1), lambda i: (i, 0)),
            pl.BlockSpec((R, EMB), lambda i: (i, 0)),
            pl.BlockSpec((R, EMB), lambda i: (i, 0)),
            pl.BlockSpec((R, EMB), lambda i: (i, 0)),
            _full((NFEAT, EMB)),
            _full((2 * EMB, EMB)), _full((1, EMB)),
            _full((2 * EMB, EMB)), _full((1, EMB)),
            _full((1, EMB)), _full((1, 1)),
            _full((EMB, NCLASS)), _full((1, NCLASS)),
        ],
        out_specs=pl.BlockSpec((R, NCLASS), lambda i: (i, 0)),
    )
    return pl.pallas_call(
        _tc_bi_body,
        grid_spec=grid_spec,
        out_shape=jax.ShapeDtypeStruct((B, NCLASS), jnp.float32),
        compiler_params=pltpu.CompilerParams(
            dimension_semantics=("arbitrary",)),
    )(xbi0, m4v, ev, parv, xs0, h0, h, wt,
      w2a, b2a, w2b, b2b, l1w, l1b, l2w, l2b)


_NC, _NS = 2, 16
_NW = _NC * _NS
_NIDX = B * NB0
_BPW = _NIDX // _NW


def _sc_gather(table2, idx):
    mesh = plsc.VectorSubcoreMesh(core_axis_name="c", subcore_axis_name="s")

    @functools.partial(
        pl.kernel, mesh=mesh,
        out_type=jax.ShapeDtypeStruct((_NIDX, 2 * EMB), jnp.float32),
        scratch_types=[
            pltpu.VMEM((_BPW,), jnp.int32),
            pltpu.VMEM((_BPW,), jnp.int32),
            pltpu.VMEM((_BPW, 2 * EMB), jnp.float32),
            pltpu.SemaphoreType.DMA,
        ],
    )
    def k(table_hbm, idx_hbm, out_hbm, idx_v, idx2_v, rows_v, sem):
        wid = lax.axis_index("s") * _NC + lax.axis_index("c")
        base = wid * _BPW
        pltpu.sync_copy(idx_hbm.at[pl.ds(base, _BPW)], idx_v)
        for j in range(_BPW // 16):
            sl = pl.ds(j * 16, 16)
            idx2_v[sl] = lax.shift_right_logical(idx_v[sl], 1)
        pltpu.async_copy(table_hbm.at[idx2_v], rows_v, sem).wait()
        pltpu.sync_copy(rows_v, out_hbm.at[pl.ds(base, _BPW)])

    return k(table2, idx)


def _sc_group_sum(x, G, row_lo=0, nrows=None):
    if nrows is None:
        nrows = x.shape[0]
    CH = 256
    rows_pt = nrows // _NW
    nchunk = rows_pt // CH
    opc = CH // G
    mesh = plsc.VectorSubcoreMesh(core_axis_name="c", subcore_axis_name="s")

    @functools.partial(
        pl.kernel, mesh=mesh,
        out_type=jax.ShapeDtypeStruct((nrows // G, NFEAT), jnp.float32),
        scratch_types=[
            pltpu.VMEM((CH, NFEAT), jnp.float32),
            pltpu.VMEM((CH, NFEAT), jnp.float32),
            pltpu.VMEM((opc, NFEAT), jnp.float32),
            pltpu.VMEM((opc, NFEAT), jnp.float32),
            pltpu.SemaphoreType.DMA,
            pltpu.SemaphoreType.DMA,
            pltpu.SemaphoreType.DMA,
            pltpu.SemaphoreType.DMA,
        ],
    )
    def k(x_hbm, o_hbm, b0, b1, ob0, ob1, s0, s1, so0, so1):
        wid = lax.axis_index("s") * _NC + lax.axis_index("c")
        base = row_lo + wid * rows_pt
        obase = wid * (rows_pt // G)
        bufs = ((b0, ob0, s0, so0), (b1, ob1, s1, so1))
        pltpu.async_copy(x_hbm.at[pl.ds(base, CH)], b0, s0)
        if nchunk > 1:
            pltpu.async_copy(x_hbm.at[pl.ds(base + CH, CH)], b1, s1)

        def step(i, carry):
            for bsel in range(2):
                buf, ob, sem, osem = bufs[bsel]
                c = 2 * i + bsel
                pltpu.make_async_copy(x_hbm.at[pl.ds(0, CH)], buf, sem).wait()

                def row_body(o, carry2):
                    for cc in range(NFEAT // 16):
                        sl = pl.ds(cc * 16, 16)
                        acc = buf[G * o, sl]
                        for j in range(1, G):
                            acc = acc + buf[G * o + j, sl]
                        ob[o, sl] = acc
                    return carry2

                @pl.when(c >= 2)
                def _():
                    pltpu.make_async_copy(
                        o_hbm.at[pl.ds(0, opc)], ob, osem).wait()

                lax.fori_loop(0, opc, row_body, 0, unroll=False)
                pltpu.async_copy(
                    ob, o_hbm.at[pl.ds(obase + c * opc, opc)], osem)

                @pl.when(c + 2 < nchunk)
                def _():
                    pltpu.async_copy(
                        x_hbm.at[pl.ds(base + (c + 2) * CH, CH)], buf, sem)
            return carry

        lax.fori_loop(0, nchunk // 2, step, 0, unroll=False)
        for bsel in range(2):
            _, ob, _, osem = bufs[bsel]
            pltpu.make_async_copy(o_hbm.at[pl.ds(0, opc)], ob, osem).wait()

    return k(x)


def kernel(x0, x1, x2, xbi0, xbi2, weight_trans, table, W1a, b1a, W1b, b1b,
           W2a, b2a, W2b, b2b, lin1_w, lin1_b, lin2_w, lin2_b, xbi1):
    m2sb = _sc_group_sum(x2, ND1, row_lo=RA * ND0 * ND1,
                         nrows=(B - RA) * ND0 * ND1)
    e = _sc_gather(table.reshape(-1, 2 * EMB), xbi1)
    m4s = _sc_group_sum(xbi2, NB1)

    x1v = x1.reshape(B, ND0, NFEAT)
    wargs = (weight_trans, W1a, b1a.reshape(1, EMB), W1b, b1b.reshape(1, EMB))
    xs0a, h0a, ha = _tc_main_a(
        x0, x1v, x2.reshape(B * ND0, ND1, NFEAT), *wargs)
    xs0b, h0b, hb = _tc_main_b(x0, x1v, m2sb, *wargs)
    xs0 = jnp.concatenate([xs0a, xs0b], axis=0)
    h0 = jnp.concatenate([h0a, h0b], axis=0)
    h = jnp.concatenate([ha, hb], axis=0)
    return _tc_bi(
        xbi0,
        m4s,
        e,
        xbi1.reshape(_NIDX, 1),
        xs0, h0, h,
        weight_trans,
        W2a, b2a.reshape(1, EMB),
        W2b, b2b.reshape(1, EMB),
        lin1_w.reshape(1, EMB), lin1_b.reshape(1, 1),
        lin2_w, lin2_b.reshape(1, NCLASS),
    )
